# baseline (speedup 1.0000x reference)
import jax
import jax.numpy as jnp
from jax.experimental import pallas as pl
from jax.experimental.pallas import tpu as pltpu

N = 2500
K = 50
T = 60
F = 5
H = 128
B = 16
G4 = 4 * H
BOUNDS = (0, 800, 1500, 2100, 2500)


def _gates(g, c):
    i = jax.nn.sigmoid(g[:, :H])
    f = jax.nn.sigmoid(g[:, H:2 * H])
    gg = jnp.tanh(g[:, 2 * H:3 * H])
    o = jax.nn.sigmoid(g[:, 3 * H:])
    c = f * c + i * gg
    h = o * jnp.tanh(c)
    return h, c


def _fused_kernel(x2d_ref, wih0_ref, whh0_ref, b0a_ref, b0b_ref,
                  wih1_ref, whh1_ref, b1a_ref, b1b_ref,
                  wp1_ref, bp1_ref, wp2_ref, bp2_ref,
                  sens_ref, thr_ref, cw_ref,
                  wi1_ref, bi1_ref, wi2_ref, bi2_ref, wi3_ref, bi3_ref,
                  whead_ref, bhead_ref,
                  heads_ref, overall_ref, acts_ref,
                  xp_ref):
    f32 = jnp.float32
    b0 = b0a_ref[:] + b0b_ref[:]
    b1 = b1a_ref[:] + b1b_ref[:]
    wih0T = jnp.transpose(wih0_ref[:])
    whh0 = jnp.transpose(whh0_ref[:])
    w1 = jnp.concatenate([jnp.transpose(wih1_ref[:]),
                          jnp.transpose(whh1_ref[:])], axis=0)
    xp_ref[:] = jnp.dot(x2d_ref[:], wih0T, preferred_element_type=f32) + b0

    def l0_step(t, h0, c0):
        g0 = xp_ref[pl.ds(t * B, B), :] + jnp.dot(h0, whh0, preferred_element_type=f32)
        return _gates(g0, c0)

    def l1_step(h0, h1, c1):
        hcat = jnp.concatenate([h0, h1], axis=1)
        g1 = jnp.dot(hcat, w1, preferred_element_type=f32) + b1
        return _gates(g1, c1)

    z = jnp.zeros((B, H), f32)
    h0, c0 = _gates(xp_ref[0:B, :], z)

    def step(t, carry):
        h0, c0, h1, c1 = carry
        nh0, nc0 = l0_step(t + 1, h0, c0)
        h1, c1 = l1_step(h0, h1, c1)
        return nh0, nc0, h1, c1

    h0, c0, h1, c1 = jax.lax.fori_loop(0, T - 1, step, (h0, c0, z, z))
    h1, c1 = l1_step(h0, h1, c1)

    h1T = jnp.transpose(h1)
    pT = jax.nn.relu(jnp.dot(wp1_ref[:], h1T, preferred_element_type=f32)
                     + bp1_ref[:])
    baseT = jnp.tanh(jnp.dot(wp2_ref[:], pT, preferred_element_type=f32)
                     + bp2_ref[:])

    ST = jnp.sum(baseT, axis=0, keepdims=True)
    Cc = jnp.dot(cw_ref[:], jnp.ones((K, 1), f32),
                 preferred_element_type=f32)
    sT = (Cc * sens_ref[:]) * ST
    smT = sT - thr_ref[:]
    nidx = jax.lax.broadcasted_iota(jnp.int32, (N, 1), 0)
    neuronT = jnp.where(nidx < BOUNDS[1], jax.nn.sigmoid(smT),
              jnp.where(nidx < BOUNDS[2], jnp.tanh(sT),
              jnp.where(nidx < BOUNDS[3], jax.nn.relu(smT),
                        jax.nn.sigmoid(sT))))

    h = jax.nn.relu(jnp.dot(wi1_ref[:], neuronT, preferred_element_type=f32)
                    + bi1_ref[:])
    h = jax.nn.relu(jnp.dot(wi2_ref[:], h, preferred_element_type=f32)
                    + bi2_ref[:])
    integT = jnp.tanh(jnp.dot(wi3_ref[:], h, preferred_element_type=f32)
                      + bi3_ref[:])
    headsT = jnp.dot(whead_ref[:], integT, preferred_element_type=f32) \
        + bhead_ref[:]
    heads_ref[:] = headsT
    overall_ref[:] = jax.nn.sigmoid(headsT[14:15, :])

    cols = []
    for j in range(4):
        lo, hi = BOUNDS[j], BOUNDS[j + 1]
        m = (nidx >= lo) & (nidx < hi)
        cols.append(jnp.sum(jnp.where(m, neuronT, 0.0), axis=0, keepdims=True)
                    * (1.0 / (hi - lo)))
    acts_ref[:] = jnp.concatenate(cols, axis=0)


def kernel(x, W_ih0, W_hh0, b_ih0, b_hh0, W_ih1, W_hh1, b_ih1, b_hh1,
           Wp1, bp1, Wp2, bp2, sens, thr, conn_w, conn_idx,
           Wi1, bi1, Wi2, bi2, Wi3, bi3, Wt, bt, Wpat, bpat,
           Wk, bk, Wv, bv, Ws, bs):
    f32 = jnp.float32
    row = lambda v: v.reshape(1, -1)
    col = lambda v: v.reshape(-1, 1)
    x2d = jnp.transpose(x, (1, 0, 2)).reshape(T * B, F)
    whead = jnp.concatenate([Wt, Wpat, Wk, Wv, Ws], axis=0)
    bhead = jnp.concatenate([bt, bpat, bk, bv, bs]).reshape(-1, 1)

    headsT, overallT, actsT = pl.pallas_call(
        _fused_kernel,
        out_shape=[
            jax.ShapeDtypeStruct((15, B), f32),
            jax.ShapeDtypeStruct((1, B), f32),
            jax.ShapeDtypeStruct((4, B), f32),
        ],
        scratch_shapes=[pltpu.VMEM((T * B, G4), f32)],
    )(x2d, W_ih0, W_hh0, row(b_ih0), row(b_hh0),
      W_ih1, W_hh1, row(b_ih1), row(b_hh1),
      Wp1, col(bp1), Wp2, col(bp2),
      col(sens), col(thr), conn_w,
      Wi1, col(bi1), Wi2, col(bi2), Wi3, col(bi3),
      whead, bhead)

    heads = headsT.T
    trend = heads[:, 0:3]
    pattern = heads[:, 3:9]
    key_levels = heads[:, 9:13]
    vol = heads[:, 13:14]
    conf = heads[:, 14:15]
    overall1 = overallT[0, :]
    return (trend, pattern, key_levels, vol, conf, overall1,
            actsT[0, :], actsT[1, :], actsT[2, :], actsT[3, :])


# speedup vs baseline: 1.1602x; 1.1602x over previous
import jax
import jax.numpy as jnp
from jax.experimental import pallas as pl
from jax.experimental.pallas import tpu as pltpu

N = 2500
K = 50
T = 60
F = 5
H = 128
B = 16
G4 = 4 * H
BOUNDS = (0, 800, 1500, 2100, 2500)


def _gates(g, c):
    i = jax.nn.sigmoid(g[:, :H])
    f = jax.nn.sigmoid(g[:, H:2 * H])
    gg = jnp.tanh(g[:, 2 * H:3 * H])
    o = jax.nn.sigmoid(g[:, 3 * H:])
    c = f * c + i * gg
    h = o * jnp.tanh(c)
    return h, c


def _fused_kernel(xw_ref, lw_ref, lb_ref, tp_ref, tb_ref, big_ref, wi1_ref,
                  out_ref, xp_ref):
    f32 = jnp.float32
    b0 = lb_ref[0:1, :] + lb_ref[1:2, :]
    b1 = lb_ref[2:3, :] + lb_ref[3:4, :]
    wih0T = jnp.transpose(xw_ref[T * B:, :])
    whh0 = jnp.transpose(lw_ref[0:G4, :])
    w1 = jnp.concatenate([jnp.transpose(lw_ref[G4:2 * G4, :]),
                          jnp.transpose(lw_ref[2 * G4:, :])], axis=0)
    xp_ref[:] = jnp.dot(xw_ref[0:T * B, :], wih0T, preferred_element_type=f32) + b0

    def l0_step(t, h0, c0):
        g0 = xp_ref[pl.ds(t * B, B), :] + jnp.dot(h0, whh0, preferred_element_type=f32)
        return _gates(g0, c0)

    def l1_step(h0, h1, c1):
        hcat = jnp.concatenate([h0, h1], axis=1)
        g1 = jnp.dot(hcat, w1, preferred_element_type=f32) + b1
        return _gates(g1, c1)

    z = jnp.zeros((B, H), f32)
    h0, c0 = _gates(xp_ref[0:B, :], z)

    def step(t, carry):
        h0, c0, h1, c1 = carry
        nh0, nc0 = l0_step(t + 1, h0, c0)
        h1, c1 = l1_step(h0, h1, c1)
        return nh0, nc0, h1, c1

    h0, c0, h1, c1 = jax.lax.fori_loop(0, T - 1, step, (h0, c0, z, z))
    h1, c1 = l1_step(h0, h1, c1)

    wp1 = tp_ref[0:64, 0:H]
    wp2 = tp_ref[64:96, 0:64]
    wi2 = tp_ref[96:160, :]
    wi3 = tp_ref[160:192, 0:64]
    whead = tp_ref[192:207, 0:32]
    bp1 = tb_ref[0:64, :]
    bp2 = tb_ref[64:96, :]
    bi1 = tb_ref[96:352, :]
    bi2 = tb_ref[352:416, :]
    bi3 = tb_ref[416:448, :]
    bhead = tb_ref[448:463, :]

    h1T = jnp.transpose(h1)
    pT = jax.nn.relu(jnp.dot(wp1, h1T, preferred_element_type=f32) + bp1)
    baseT = jnp.tanh(jnp.dot(wp2, pT, preferred_element_type=f32) + bp2)

    ST = jnp.sum(baseT, axis=0, keepdims=True)
    Cc = jnp.sum(big_ref[:, 0:K], axis=1, keepdims=True)
    sens = big_ref[:, K:K + 1]
    thr = big_ref[:, K + 1:K + 2]
    sT = (Cc * sens) * ST
    smT = sT - thr
    nidx = jax.lax.broadcasted_iota(jnp.int32, (N, 1), 0)
    neuronT = jnp.where(nidx < BOUNDS[1], jax.nn.sigmoid(smT),
              jnp.where(nidx < BOUNDS[2], jnp.tanh(sT),
              jnp.where(nidx < BOUNDS[3], jax.nn.relu(smT),
                        jax.nn.sigmoid(sT))))

    h = jax.nn.relu(jnp.dot(wi1_ref[:], neuronT, preferred_element_type=f32)
                    + bi1)
    h = jax.nn.relu(jnp.dot(wi2, h, preferred_element_type=f32) + bi2)
    integT = jnp.tanh(jnp.dot(wi3, h, preferred_element_type=f32) + bi3)
    headsT = jnp.dot(whead, integT, preferred_element_type=f32) + bhead
    out_ref[0:15, :] = headsT
    out_ref[15:16, :] = jax.nn.sigmoid(headsT[14:15, :])

    cols = []
    for j in range(4):
        lo, hi = BOUNDS[j], BOUNDS[j + 1]
        m = (nidx >= lo) & (nidx < hi)
        cols.append(jnp.sum(jnp.where(m, neuronT, 0.0), axis=0, keepdims=True)
                    * (1.0 / (hi - lo)))
    out_ref[16:20, :] = jnp.concatenate(cols, axis=0)


def kernel(x, W_ih0, W_hh0, b_ih0, b_hh0, W_ih1, W_hh1, b_ih1, b_hh1,
           Wp1, bp1, Wp2, bp2, sens, thr, conn_w, conn_idx,
           Wi1, bi1, Wi2, bi2, Wi3, bi3, Wt, bt, Wpat, bpat,
           Wk, bk, Wv, bv, Ws, bs):
    f32 = jnp.float32
    pad256 = lambda w: jnp.pad(w, ((0, 0), (0, 256 - w.shape[1])))
    x2d = jnp.transpose(x, (1, 0, 2)).reshape(T * B, F)
    xw = jnp.concatenate([x2d, W_ih0], axis=0)
    lw = jnp.concatenate([W_hh0, W_ih1, W_hh1], axis=0)
    lb = jnp.stack([b_ih0, b_hh0, b_ih1, b_hh1], axis=0)
    whead = jnp.concatenate([Wt, Wpat, Wk, Wv, Ws], axis=0)
    tp = jnp.concatenate([pad256(Wp1), pad256(Wp2), Wi2,
                          pad256(Wi3), pad256(whead)], axis=0)
    tb = jnp.concatenate([bp1, bp2, bi1, bi2, bi3,
                          bt, bpat, bk, bv, bs]).reshape(-1, 1)
    big = jnp.concatenate([conn_w, sens.reshape(-1, 1),
                           thr.reshape(-1, 1)], axis=1)

    out = pl.pallas_call(
        _fused_kernel,
        out_shape=jax.ShapeDtypeStruct((20, B), f32),
        scratch_shapes=[pltpu.VMEM((T * B, G4), f32)],
    )(xw, lw, lb, tp, tb, big, Wi1)

    heads = out[0:15, :].T
    trend = heads[:, 0:3]
    pattern = heads[:, 3:9]
    key_levels = heads[:, 9:13]
    vol = heads[:, 13:14]
    conf = heads[:, 14:15]
    overall1 = out[15, :]
    return (trend, pattern, key_levels, vol, conf, overall1,
            out[16, :], out[17, :], out[18, :], out[19, :])


# speedup vs baseline: 1.4140x; 1.2187x over previous
import jax
import jax.numpy as jnp
from jax.experimental import pallas as pl
from jax.experimental.pallas import tpu as pltpu

N = 2500
K = 50
T = 60
F = 5
H = 128
B = 16
G4 = 4 * H
BOUNDS = (0, 800, 1500, 2100, 2500)


def _gates(g, c):
    i = jax.nn.sigmoid(g[:, :H])
    f = jax.nn.sigmoid(g[:, H:2 * H])
    gg = jnp.tanh(g[:, 2 * H:3 * H])
    o = jax.nn.sigmoid(g[:, 3 * H:])
    c = f * c + i * gg
    h = o * jnp.tanh(c)
    return h, c


def _fused_kernel(xw_ref, lw_ref, lb_ref, tp_ref, tb_ref, big_ref, wi1_ref,
                  out_ref, xp_ref):
    f32 = jnp.float32
    b0 = lb_ref[0:1, :] + lb_ref[1:2, :]
    b1 = lb_ref[2:3, :] + lb_ref[3:4, :]
    wih0T = jnp.transpose(xw_ref[T * B:, :])
    whh0 = jnp.transpose(lw_ref[0:G4, :])
    w1 = jnp.concatenate([jnp.transpose(lw_ref[G4:2 * G4, :]),
                          jnp.transpose(lw_ref[2 * G4:, :])], axis=0)
    xp_ref[:] = jnp.dot(xw_ref[0:T * B, :], wih0T, preferred_element_type=f32) + b0

    def l0_step(t, h0, c0):
        g0 = xp_ref[pl.ds(t * B, B), :] + jnp.dot(h0, whh0, preferred_element_type=f32)
        return _gates(g0, c0)

    def l1_step(h0, h1, c1):
        hcat = jnp.concatenate([h0, h1], axis=1)
        g1 = jnp.dot(hcat, w1, preferred_element_type=f32) + b1
        return _gates(g1, c1)

    z = jnp.zeros((B, H), f32)
    h0, c0 = _gates(xp_ref[0:B, :], z)

    def step(t, carry):
        h0, c0, h1, c1 = carry
        nh0, nc0 = l0_step(t + 1, h0, c0)
        h1, c1 = l1_step(h0, h1, c1)
        return nh0, nc0, h1, c1

    h0, c0, h1, c1 = jax.lax.fori_loop(0, T - 1, step, (h0, c0, z, z))
    h1, c1 = l1_step(h0, h1, c1)

    wp1 = tp_ref[0:64, 0:H]
    wp2 = tp_ref[64:96, 0:64]
    wi2 = tp_ref[96:160, :]
    wi3 = tp_ref[160:192, 0:64]
    whead = tp_ref[192:207, 0:32]
    bp1 = tb_ref[0:64, :]
    bp2 = tb_ref[64:96, :]
    bi1 = tb_ref[96:352, :]
    bi2 = tb_ref[352:416, :]
    bi3 = tb_ref[416:448, :]
    bhead = tb_ref[448:463, :]

    h1T = jnp.transpose(h1)
    pT = jax.nn.relu(jnp.dot(wp1, h1T, preferred_element_type=f32) + bp1)
    baseT = jnp.tanh(jnp.dot(wp2, pT, preferred_element_type=f32) + bp2)

    ST = jnp.sum(baseT, axis=0, keepdims=True)
    sens = big_ref[:, 126:127]
    thr = big_ref[:, 127:128]
    Cc = jnp.sum(big_ref[:], axis=1, keepdims=True) - sens - thr
    sT = (Cc * sens) * ST
    smT = sT - thr
    nidx = jax.lax.broadcasted_iota(jnp.int32, (N, 1), 0)
    neuronT = jnp.where(nidx < BOUNDS[1], jax.nn.sigmoid(smT),
              jnp.where(nidx < BOUNDS[2], jnp.tanh(sT),
              jnp.where(nidx < BOUNDS[3], jax.nn.relu(smT),
                        jax.nn.sigmoid(sT))))

    h = jax.nn.relu(jnp.dot(wi1_ref[:], neuronT, preferred_element_type=f32)
                    + bi1)
    h = jax.nn.relu(jnp.dot(wi2, h, preferred_element_type=f32) + bi2)
    integT = jnp.tanh(jnp.dot(wi3, h, preferred_element_type=f32) + bi3)
    headsT = jnp.dot(whead, integT, preferred_element_type=f32) + bhead
    out_ref[0:15, :] = headsT
    out_ref[15:16, :] = jax.nn.sigmoid(headsT[14:15, :])

    cols = []
    for j in range(4):
        lo, hi = BOUNDS[j], BOUNDS[j + 1]
        m = (nidx >= lo) & (nidx < hi)
        cols.append(jnp.sum(jnp.where(m, neuronT, 0.0), axis=0, keepdims=True)
                    * (1.0 / (hi - lo)))
    out_ref[16:20, :] = jnp.concatenate(cols, axis=0)


def kernel(x, W_ih0, W_hh0, b_ih0, b_hh0, W_ih1, W_hh1, b_ih1, b_hh1,
           Wp1, bp1, Wp2, bp2, sens, thr, conn_w, conn_idx,
           Wi1, bi1, Wi2, bi2, Wi3, bi3, Wt, bt, Wpat, bpat,
           Wk, bk, Wv, bv, Ws, bs):
    f32 = jnp.float32
    pad256 = lambda w: jnp.pad(w, ((0, 0), (0, 256 - w.shape[1])))
    x2d = jnp.transpose(x, (1, 0, 2)).reshape(T * B, F)
    xw = jnp.concatenate([x2d, W_ih0], axis=0)
    lw = jnp.concatenate([W_hh0, W_ih1, W_hh1], axis=0)
    lb = jnp.stack([b_ih0, b_hh0, b_ih1, b_hh1], axis=0)
    whead = jnp.concatenate([Wt, Wpat, Wk, Wv, Ws], axis=0)
    tp = jnp.concatenate([pad256(Wp1), pad256(Wp2), Wi2,
                          pad256(Wi3), pad256(whead)], axis=0)
    tb = jnp.concatenate([bp1, bp2, bi1, bi2, bi3,
                          bt, bpat, bk, bv, bs]).reshape(-1, 1)
    big = jnp.concatenate([conn_w, jnp.zeros((N, 126 - K), f32),
                           sens.reshape(-1, 1),
                           thr.reshape(-1, 1)], axis=1)

    out = pl.pallas_call(
        _fused_kernel,
        out_shape=jax.ShapeDtypeStruct((20, B), f32),
        scratch_shapes=[pltpu.VMEM((T * B, G4), f32)],
    )(xw, lw, lb, tp, tb, big, Wi1)

    heads = out[0:15, :].T
    trend = heads[:, 0:3]
    pattern = heads[:, 3:9]
    key_levels = heads[:, 9:13]
    vol = heads[:, 13:14]
    conf = heads[:, 14:15]
    overall1 = out[15, :]
    return (trend, pattern, key_levels, vol, conf, overall1,
            out[16, :], out[17, :], out[18, :], out[19, :])
